# baseline (speedup 1.0000x reference)
import math

import jax
import jax.numpy as jnp
import numpy as np
from jax.experimental import pallas as pl
from jax.experimental.pallas import tpu as pltpu

_B = 16
_IN_DIM = 3
_POS_PROJ = 6
_HID = 64
_OUT = 4
_PE = _IN_DIM * 2 * _POS_PROJ
_D_IN = _IN_DIM + _PE
_DIMS = [(_D_IN, _HID), (_HID, _HID), (_HID, _OUT)]
_BLK = 2048
_CHK = 512

_F32 = jnp.float32
_PREC = jax.lax.Precision.DEFAULT
_PREC_HI = jax.lax.Precision.HIGHEST


def _dt(a, w, prec=_PREC):
    return jax.lax.dot_general(
        a, w, (((1,), (1,)), ((), ())),
        preferred_element_type=_F32, precision=prec)


def _dm(w, a):
    return jax.lax.dot_general(
        w, a, (((1,), (0,)), ((), ())),
        preferred_element_type=_F32, precision=_PREC)


def _ln(h):
    m = jnp.mean(h, axis=-1, keepdims=True)
    v = jnp.mean((h - m) ** 2, axis=-1, keepdims=True)
    return (h - m) * jax.lax.rsqrt(v + 1e-5)


def _ln0(h):
    m = jnp.mean(h, axis=0, keepdims=True)
    d = h - m
    v = jnp.mean(d * d, axis=0, keepdims=True)
    return d * jax.lax.rsqrt(v + 1e-5)


def _body(i_ref, x_ref, c_ref, *rest):
    p = rest[:30]
    o_ref = rest[30]
    f_ref = rest[31]
    wb = rest[32:]
    t = pl.program_id(0)

    @pl.when(t == 0)
    def _hyper():
        cc = c_ref[...]
        for l, (din, dout) in enumerate(_DIMS):
            w0, b0, g0, be0, w1, b1, g1, be1, w2, b2 = p[l * 10:(l + 1) * 10]
            wref, bref = wb[2 * l], wb[2 * l + 1]
            h = _dt(cc, w0[...]) + b0[...]
            h = jnp.maximum(_ln(h) * g0[...] + be0[...], 0.0)
            h = _dt(h, w1[...]) + b1[...]
            h = jnp.maximum(_ln(h) * g1[...] + be1[...], 0.0)
            for o in range(dout):
                wsl = w2[o * din:(o + 1) * din, :]
                bsl = b2[0:1, o * din:(o + 1) * din]
                wref[:, o, :] = _dt(h, wsl[...]) + bsl[...]
            bT = _dt(w2[din * dout:din * dout + dout, :], h)
            eye = (jax.lax.broadcasted_iota(jnp.int32, (dout, dout), 0) ==
                   jax.lax.broadcasted_iota(jnp.int32, (dout, dout), 1)
                   ).astype(_F32)
            bcol = _dt(eye, b2[0:1, din * dout:din * dout + dout],
                       prec=_PREC_HI)
            for s in range(_B):
                bref[s] = bT[:, s:s + 1] + bcol

    w0s, b0s, w1s, b1s, w2s, b2s = wb

    xt = x_ref[...]
    f_ref[0:_IN_DIM, :] = xt
    s = jnp.sin(jnp.float32(math.pi) * xt)
    c = jnp.cos(jnp.float32(math.pi) * xt)
    for j in range(_POS_PROJ):
        for d in range(_IN_DIM):
            base = _IN_DIM + d * 2 * _POS_PROJ + j
            f_ref[base:base + 1, :] = s[d:d + 1, :]
            f_ref[base + _POS_PROJ:base + _POS_PROJ + 1, :] = c[d:d + 1, :]
        if j < _POS_PROJ - 1:
            s, c = 2.0 * s * c, 1.0 - 2.0 * s * s
    feats = f_ref[...]

    iv = i_ref[0]

    for ci in range(_BLK // _CHK):
        lo = ci * _CHK
        fch = feats[:, lo:lo + _CHK]
        ivc = iv[:, lo:lo + _CHK]
        s_lo = i_ref[0, 0, lo]
        s_hi = i_ref[0, 0, lo + _CHK - 1]

        def seg_body(seg, acc, fch=fch, ivc=ivc):
            h = _dm(w0s[seg], fch) + b0s[seg]
            h = jnp.maximum(_ln0(h), 0.0)
            h = _dm(w1s[seg], h) + b1s[seg]
            h = jnp.maximum(_ln0(h), 0.0)
            o = _dm(w2s[seg], h) + b2s[seg]
            return acc + jnp.where(ivc == seg, o, 0.0)

        o_ref[:, lo:lo + _CHK] = jax.lax.fori_loop(
            s_lo, s_hi + 1, seg_body, jnp.zeros((_OUT, _CHK), _F32))


def kernel(x, i, c, params):
    n = x.shape[0]
    b = c.shape[0]
    t = n // _BLK

    plist = []
    for l in range(3):
        for name in ("W0", "b0", "g0", "be0", "W1", "b1", "g1", "be1", "W2", "b2"):
            pa = params[f"h{l}_{name}"]
            plist.append(pa.reshape(1, -1) if pa.ndim == 1 else pa)

    ii = i.astype(jnp.int32)
    i3 = ii.reshape(t, 1, _BLK)
    xt = x.T

    full = lambda shape: pl.BlockSpec(shape, lambda tt: (0,) * len(shape))
    out = pl.pallas_call(
        _body,
        grid=(t,),
        in_specs=[
            pl.BlockSpec((1, 1, _BLK), lambda tt: (tt, 0, 0)),
            pl.BlockSpec((_IN_DIM, _BLK), lambda tt: (0, tt)),
            full(c.shape),
        ] + [full(pa.shape) for pa in plist],
        out_specs=pl.BlockSpec((_OUT, _BLK), lambda tt: (0, tt)),
        scratch_shapes=[
            pltpu.VMEM((_D_IN, _BLK), _F32),
            pltpu.VMEM((b, _HID, _D_IN), _F32),
            pltpu.VMEM((b, _HID, 1), _F32),
            pltpu.VMEM((b, _HID, _HID), _F32),
            pltpu.VMEM((b, _HID, 1), _F32),
            pltpu.VMEM((b, _OUT, _HID), _F32),
            pltpu.VMEM((b, _OUT, 1), _F32),
        ],
        out_shape=jax.ShapeDtypeStruct((_OUT, n), _F32),
    )(i3, xt, c, *plist)
    return out.T


# speedup vs baseline: 1.5714x; 1.5714x over previous
import math

import jax
import jax.numpy as jnp
import numpy as np
from jax.experimental import pallas as pl
from jax.experimental.pallas import tpu as pltpu

_B = 16
_IN_DIM = 3
_POS_PROJ = 6
_HID = 64
_OUT = 4
_PE = _IN_DIM * 2 * _POS_PROJ
_D_IN = _IN_DIM + _PE
_DIMS = [(_D_IN, _HID), (_HID, _HID), (_HID, _OUT)]
_BLK = 2048

_F32 = jnp.float32
_PREC = jax.lax.Precision.DEFAULT
_PREC_HI = jax.lax.Precision.HIGHEST


def _dt(a, w, prec=_PREC):
    return jax.lax.dot_general(
        a, w, (((1,), (1,)), ((), ())),
        preferred_element_type=_F32, precision=prec)


def _dm(w, a):
    return jax.lax.dot_general(
        w, a, (((1,), (0,)), ((), ())),
        preferred_element_type=_F32, precision=_PREC)


def _ln(h):
    m = jnp.mean(h, axis=-1, keepdims=True)
    v = jnp.mean((h - m) ** 2, axis=-1, keepdims=True)
    return (h - m) * jax.lax.rsqrt(v + 1e-5)


def _ln0(h):
    m = jnp.mean(h, axis=0, keepdims=True)
    d = h - m
    v = jnp.mean(d * d, axis=0, keepdims=True)
    return d * jax.lax.rsqrt(v + 1e-5)


def _body(i_ref, x_ref, c_ref, *rest):
    p = rest[:30]
    o_ref = rest[30]
    f_ref = rest[31]
    wb = rest[32:]
    t = pl.program_id(0)

    @pl.when(t == 0)
    def _hyper():
        cc = c_ref[...]
        for l, (din, dout) in enumerate(_DIMS):
            w0, b0, g0, be0, w1, b1, g1, be1, w2, b2 = p[l * 10:(l + 1) * 10]
            wref, bref = wb[2 * l], wb[2 * l + 1]
            h = _dt(cc, w0[...]) + b0[...]
            h = jnp.maximum(_ln(h) * g0[...] + be0[...], 0.0)
            h = _dt(h, w1[...]) + b1[...]
            h = jnp.maximum(_ln(h) * g1[...] + be1[...], 0.0)
            for o in range(dout):
                wsl = w2[o * din:(o + 1) * din, :]
                bsl = b2[0:1, o * din:(o + 1) * din]
                wref[:, o, :] = _dt(h, wsl[...]) + bsl[...]
            bT = _dt(w2[din * dout:din * dout + dout, :], h)
            eye = (jax.lax.broadcasted_iota(jnp.int32, (dout, dout), 0) ==
                   jax.lax.broadcasted_iota(jnp.int32, (dout, dout), 1)
                   ).astype(_F32)
            bcol = _dt(eye, b2[0:1, din * dout:din * dout + dout],
                       prec=_PREC_HI)
            for s in range(_B):
                bref[s] = bT[:, s:s + 1] + bcol

    w0s, b0s, w1s, b1s, w2s, b2s = wb

    xt = x_ref[...]
    f_ref[0:_IN_DIM, :] = xt
    s = jnp.sin(jnp.float32(math.pi) * xt)
    c = jnp.cos(jnp.float32(math.pi) * xt)
    for j in range(_POS_PROJ):
        for d in range(_IN_DIM):
            base = _IN_DIM + d * 2 * _POS_PROJ + j
            f_ref[base:base + 1, :] = s[d:d + 1, :]
            f_ref[base + _POS_PROJ:base + _POS_PROJ + 1, :] = c[d:d + 1, :]
        if j < _POS_PROJ - 1:
            s, c = 2.0 * s * c, 1.0 - 2.0 * s * s
    feats = f_ref[...]

    iv = i_ref[0]
    s_lo = i_ref[0, 0, 0]
    s_hi = i_ref[0, 0, _BLK - 1]

    def seg_body(seg, acc):
        h = _dm(w0s[seg], feats) + b0s[seg]
        h = jnp.maximum(_ln0(h), 0.0)
        h = _dm(w1s[seg], h) + b1s[seg]
        h = jnp.maximum(_ln0(h), 0.0)
        o = _dm(w2s[seg], h) + b2s[seg]
        return acc + jnp.where(iv == seg, o, 0.0)

    o_ref[...] = jax.lax.fori_loop(
        s_lo, s_hi + 1, seg_body, jnp.zeros((_OUT, _BLK), _F32))


def kernel(x, i, c, params):
    n = x.shape[0]
    b = c.shape[0]
    t = n // _BLK

    plist = []
    for l in range(3):
        for name in ("W0", "b0", "g0", "be0", "W1", "b1", "g1", "be1", "W2", "b2"):
            pa = params[f"h{l}_{name}"]
            plist.append(pa.reshape(1, -1) if pa.ndim == 1 else pa)

    ii = i.astype(jnp.int32)
    i3 = ii.reshape(t, 1, _BLK)
    xt = x.T

    full = lambda shape: pl.BlockSpec(shape, lambda tt: (0,) * len(shape))
    out = pl.pallas_call(
        _body,
        grid=(t,),
        in_specs=[
            pl.BlockSpec((1, 1, _BLK), lambda tt: (tt, 0, 0)),
            pl.BlockSpec((_IN_DIM, _BLK), lambda tt: (0, tt)),
            full(c.shape),
        ] + [full(pa.shape) for pa in plist],
        out_specs=pl.BlockSpec((_OUT, _BLK), lambda tt: (0, tt)),
        scratch_shapes=[
            pltpu.VMEM((_D_IN, _BLK), _F32),
            pltpu.VMEM((b, _HID, _D_IN), _F32),
            pltpu.VMEM((b, _HID, 1), _F32),
            pltpu.VMEM((b, _HID, _HID), _F32),
            pltpu.VMEM((b, _HID, 1), _F32),
            pltpu.VMEM((b, _OUT, _HID), _F32),
            pltpu.VMEM((b, _OUT, 1), _F32),
        ],
        out_shape=jax.ShapeDtypeStruct((_OUT, n), _F32),
    )(i3, xt, c, *plist)
    return out.T
